# baseline (speedup 1.0000x reference)
import functools

import jax
import jax.numpy as jnp
from jax import lax
from jax.experimental import pallas as pl
from jax.experimental.pallas import tpu as pltpu
from jax.experimental.pallas import tpu_sc as plsc

BASE = 10000.0
ROW = 1536
NC, NS = 2, 16
NW = NC * NS
CH = 16


def _sc_rope(tabc_hbm, tabs_hbm, pt_hbm, py_hbm, px_hbm, tok_hbm, out_hbm,
             tabc_v, tabs_v, pt_v, py_v, px_v, x_v):
    wid = lax.axis_index("s") * NC + lax.axis_index("c")
    tpw = tok_hbm.shape[0] // NW
    base = wid * tpw
    pltpu.sync_copy(tabc_hbm, tabc_v)
    pltpu.sync_copy(tabs_hbm, tabs_v)
    pltpu.sync_copy(pt_hbm.at[pl.ds(base, tpw)], pt_v)
    pltpu.sync_copy(py_hbm.at[pl.ds(base, tpw)], py_v)
    pltpu.sync_copy(px_hbm.at[pl.ds(base, tpw)], px_v)
    iota = lax.iota(jnp.int32, 16)

    def chunk_body(g, carry):
        pltpu.sync_copy(tok_hbm.at[pl.ds(base + g * CH, CH)], x_v)

        def tok_body(t, carry2):
            tg = jnp.full((16,), g * CH + t, jnp.int32)
            bpt = plsc.load_gather(pt_v, [tg])
            bpy = plsc.load_gather(py_v, [tg]) + 16
            bpx = plsc.load_gather(px_v, [tg]) + 48
            cs = [(plsc.load_gather(tabc_v, [b, iota]),
                   plsc.load_gather(tabs_v, [b, iota]))
                  for b in (bpt, bpy, bpx)]

            for h in range(16):
                for sidx, (c, s) in enumerate(cs):
                    o = h * 96 + sidx * 32
                    x0 = x_v[t, pl.ds(o, 16)]
                    x1 = x_v[t, pl.ds(o + 16, 16)]
                    x_v[t, pl.ds(o, 16)] = x0 * c + x1 * s
                    x_v[t, pl.ds(o + 16, 16)] = x1 * c + x0 * s
            return carry2

        lax.fori_loop(0, CH, tok_body, 0)
        pltpu.sync_copy(x_v, out_hbm.at[pl.ds(base + g * CH, CH)])
        return carry

    lax.fori_loop(0, tpw // CH, chunk_body, 0)


def _build_sc_tables():
    inv = 1.0 / BASE ** (jnp.arange(16, dtype=jnp.float32) / 16.0)
    th = jnp.concatenate([
        jnp.arange(16, dtype=jnp.float32),
        jnp.arange(32, dtype=jnp.float32),
        jnp.arange(32, dtype=jnp.float32)])[:, None] * inv[None, :]
    return jnp.cos(th), jnp.sin(th)


@jax.jit
def kernel(tokens, pos_t, pos_y, pos_x):
    B, N, H, dim = tokens.shape
    M = B * N
    tok2 = tokens.reshape(M, H * dim)
    tabc, tabs = _build_sc_tables()
    mesh = plsc.VectorSubcoreMesh(core_axis_name="c", subcore_axis_name="s")
    sck = functools.partial(
        pl.kernel,
        mesh=mesh,
        compiler_params=pltpu.CompilerParams(needs_layout_passes=False),
        out_type=jax.ShapeDtypeStruct((M, ROW), jnp.float32),
        scratch_types=[
            pltpu.VMEM((80, 16), jnp.float32),
            pltpu.VMEM((80, 16), jnp.float32),
            pltpu.VMEM((M // NW,), jnp.int32),
            pltpu.VMEM((M // NW,), jnp.int32),
            pltpu.VMEM((M // NW,), jnp.int32),
            pltpu.VMEM((CH, ROW), jnp.float32),
        ],
    )(_sc_rope)
    out = sck(tabc, tabs, pos_t.reshape(M), pos_y.reshape(M),
              pos_x.reshape(M), tok2)
    return out.reshape(B, N, H, dim)


# speedup vs baseline: 1.1096x; 1.1096x over previous
import jax
import jax.numpy as jnp
from jax.experimental import pallas as pl
from jax.experimental.pallas import tpu as pltpu

BASE = 10000.0
NB = 1024
ROW = 1536
NT, NY, NX = 16, 32, 32


def _rope_kernel(pos_ref, tab_ref, tokens_ref, out_ref):
    p = pos_ref[...]
    i80 = jax.lax.broadcasted_iota(jnp.int32, (NT + NY + NX, NB), 0)
    hit = (i80 == p[0:1, :]) | (i80 == p[1:2, :] + NT) \
        | (i80 == p[2:3, :] + (NT + NY))
    oht = jnp.where(hit, 1.0, 0.0).astype(jnp.bfloat16)
    cs = jax.lax.dot_general(
        oht, tab_ref[...], (((0,), (0,)), ((), ())),
        preferred_element_type=jnp.float32)
    c = cs[None, :, :ROW]
    s = cs[None, :, ROW:]
    x = tokens_ref[...]
    rl = jnp.concatenate([x[:, :, 16:], x[:, :, :16]], axis=-1)
    rr = jnp.concatenate([x[:, :, -16:], x[:, :, :-16]], axis=-1)
    lane = jax.lax.broadcasted_iota(jnp.int32, (1, 1, ROW), 2)
    r = jnp.where(lane % 32 < 16, rl, rr)
    out_ref[...] = x * c + r * s


def _build_table(H):
    inv_freq = 1.0 / BASE ** (jnp.arange(0, 32, 2, dtype=jnp.float32) / 32.0)

    def sec_rows(n, lo, hi):
        th = jnp.arange(n, dtype=jnp.float32)[:, None] * inv_freq[None, :]
        out = []
        for f in (jnp.cos, jnp.sin):
            v = f(th)
            v32 = jnp.concatenate([v, v], axis=-1)
            row96 = jnp.concatenate(
                [jnp.zeros((n, lo), jnp.float32), v32,
                 jnp.zeros((n, hi), jnp.float32)], axis=-1)
            out.append(jnp.tile(row96, (1, H)))
        return jnp.concatenate(out, axis=-1)

    return jnp.concatenate([
        sec_rows(NT, 0, 64), sec_rows(NY, 32, 32), sec_rows(NX, 64, 0),
    ], axis=0)


@jax.jit
def kernel(tokens, pos_t, pos_y, pos_x):
    B, N, H, dim = tokens.shape
    M = B * N
    pos = jnp.stack([pos_t, pos_y, pos_x], axis=0).reshape(3, M)
    tok2 = tokens.reshape(1, M, H * dim)
    table = _build_table(H).astype(jnp.bfloat16)
    grid = (M // NB,)
    out = pl.pallas_call(
        _rope_kernel,
        grid=grid,
        in_specs=[
            pl.BlockSpec((3, NB), lambda i: (0, i)),
            pl.BlockSpec((NT + NY + NX, 2 * H * dim), lambda i: (0, 0)),
            pl.BlockSpec((1, NB, H * dim), lambda i: (0, i, 0)),
        ],
        out_specs=pl.BlockSpec((1, NB, H * dim), lambda i: (0, i, 0)),
        out_shape=jax.ShapeDtypeStruct((1, M, H * dim), tokens.dtype),
        compiler_params=pltpu.CompilerParams(
            fuse_transposed_lhs_in_matmul=True),
    )(pos, table, tok2)
    return out.reshape(B, N, H, dim)


# speedup vs baseline: 2.2040x; 1.9864x over previous
import jax
import jax.numpy as jnp
from jax.experimental import pallas as pl
from jax.experimental.pallas import tpu as pltpu

BASE = 10000.0
NB = 1024
ROW = 1536
NT, NY, NX = 16, 32, 32


def _rope_kernel(pos_ref, tab_ref, tokens_ref, out_ref):
    p = pos_ref[...]
    l80 = jax.lax.broadcasted_iota(jnp.int32, (1, NB, NT + NY + NX), 2)
    hit = (l80 == p[:, :, 0:1]) | (l80 == p[:, :, 1:2] + NT) \
        | (l80 == p[:, :, 2:3] + (NT + NY))
    oh = jnp.where(hit, 1.0, 0.0)[0].astype(jnp.bfloat16)
    cs = jax.lax.dot_general(
        oh, tab_ref[...], (((1,), (0,)), ((), ())),
        preferred_element_type=jnp.float32)
    c = cs[None, :, :ROW]
    s = cs[None, :, ROW:]
    x = tokens_ref[...]
    rl = jnp.concatenate([x[:, :, 16:], x[:, :, :16]], axis=-1)
    rr = jnp.concatenate([x[:, :, -16:], x[:, :, :-16]], axis=-1)
    lane = jax.lax.broadcasted_iota(jnp.int32, (1, 1, ROW), 2)
    r = jnp.where(lane % 32 < 16, rl, rr)
    out_ref[...] = x * c + r * s


def _build_table(H):
    inv_freq = 1.0 / BASE ** (jnp.arange(0, 32, 2, dtype=jnp.float32) / 32.0)

    def sec_rows(n, lo, hi):
        th = jnp.arange(n, dtype=jnp.float32)[:, None] * inv_freq[None, :]
        out = []
        for f in (jnp.cos, jnp.sin):
            v = f(th)
            v32 = jnp.concatenate([v, v], axis=-1)
            row96 = jnp.concatenate(
                [jnp.zeros((n, lo), jnp.float32), v32,
                 jnp.zeros((n, hi), jnp.float32)], axis=-1)
            out.append(jnp.tile(row96, (1, H)))
        return jnp.concatenate(out, axis=-1)

    return jnp.concatenate([
        sec_rows(NT, 0, 64), sec_rows(NY, 32, 32), sec_rows(NX, 64, 0),
    ], axis=0)


@jax.jit
def kernel(tokens, pos_t, pos_y, pos_x):
    B, N, H, dim = tokens.shape
    pos = jnp.stack([pos_t, pos_y, pos_x], axis=-1)
    tok2 = tokens.reshape(B, N, H * dim)
    table = _build_table(H).astype(jnp.bfloat16)
    grid = (B, N // NB)
    out = pl.pallas_call(
        _rope_kernel,
        grid=grid,
        in_specs=[
            pl.BlockSpec((1, NB, 3), lambda b, i: (b, i, 0)),
            pl.BlockSpec((NT + NY + NX, 2 * H * dim), lambda b, i: (0, 0)),
            pl.BlockSpec((1, NB, H * dim), lambda b, i: (b, i, 0)),
        ],
        out_specs=pl.BlockSpec((1, NB, H * dim), lambda b, i: (b, i, 0)),
        out_shape=jax.ShapeDtypeStruct((B, N, H * dim), tokens.dtype),
    )(pos, table, tok2)
    return out.reshape(B, N, H, dim)
